# Initial kernel scaffold; baseline (speedup 1.0000x reference)
import jax
import jax.numpy as jnp
from jax.experimental import pallas as pl


def kernel(input_ids, token_type_ids, token_table, pos_table, type_table, ln_gamma, ln_beta):
    raise NotImplementedError("write your pallas kernel here")


# speedup vs baseline: 2.0121x; 2.0121x over previous
import functools

import jax
import jax.numpy as jnp
from jax import lax
from jax.experimental import pallas as pl
from jax.experimental.pallas import tpu as pltpu
from jax.experimental.pallas import tpu_sc as plsc

_LN_EPS = 1e-5


def _sc_gather(table, ids_flat, n_tokens, hidden, chunk=64):
    info = plsc.get_sparse_core_info()
    nc, ns = info.num_cores, info.num_subcores
    nw = nc * ns
    per_w = n_tokens // nw
    n_chunks = per_w // chunk
    mesh = plsc.VectorSubcoreMesh(core_axis_name="c", subcore_axis_name="s")

    @functools.partial(
        pl.kernel,
        out_type=jax.ShapeDtypeStruct((n_tokens, hidden), jnp.float32),
        mesh=mesh,
        scratch_types=[
            pltpu.VMEM((chunk,), jnp.int32),
            pltpu.VMEM((chunk,), jnp.int32),
            pltpu.VMEM((chunk, hidden), jnp.float32),
            pltpu.VMEM((chunk, hidden), jnp.float32),
            pltpu.SemaphoreType.DMA,
            pltpu.SemaphoreType.DMA,
        ],
    )
    def gather_k(ids_hbm, table_hbm, out_hbm, idx0, idx1, rows0, rows1, sem0, sem1):
        wid = lax.axis_index("s") * nc + lax.axis_index("c")
        base = wid * per_w
        idxs = (idx0, idx1)
        rows = (rows0, rows1)
        sems = (sem0, sem1)
        pltpu.sync_copy(ids_hbm.at[pl.ds(base, chunk)], idx0)
        copies = [pltpu.async_copy(table_hbm.at[idx0], rows0, sem0)]
        for it in range(n_chunks):
            cur = it % 2
            nxt = (it + 1) % 2
            if it + 1 < n_chunks:
                pltpu.sync_copy(
                    ids_hbm.at[pl.ds(base + (it + 1) * chunk, chunk)], idxs[nxt]
                )
                copies.append(
                    pltpu.async_copy(table_hbm.at[idxs[nxt]], rows[nxt], sems[nxt])
                )
            copies[it].wait()
            pltpu.sync_copy(rows[cur], out_hbm.at[pl.ds(base + it * chunk, chunk)])

    return gather_k(ids_flat, table)


def _ln_body(tok_ref, tid_ref, pos_ref, typ_ref, g_ref, b_ref, out_ref):
    x = tok_ref[0]
    tid = tid_ref[0, 0]
    t0 = typ_ref[0]
    t1 = typ_ref[1]
    m = (tid == 1).astype(jnp.float32)[:, None]
    x = x + pos_ref[...] + t0[None, :] + m * (t1 - t0)[None, :]
    mean = jnp.mean(x, axis=-1, keepdims=True)
    xc = x - mean
    var = jnp.mean(xc * xc, axis=-1, keepdims=True)
    y = xc * lax.rsqrt(var + _LN_EPS)
    out_ref[0] = y * g_ref[0][None, :] + b_ref[0][None, :]


def kernel(input_ids, token_type_ids, token_table, pos_table, type_table, ln_gamma, ln_beta):
    b, s = input_ids.shape
    _, h = token_table.shape
    n = b * s
    ids_flat = input_ids.reshape(n).astype(jnp.int32)
    tok_flat = _sc_gather(token_table, ids_flat, n, h)
    tok = tok_flat.reshape(b, s, h)

    tb = 512
    nj = s // tb
    tid3 = token_type_ids.astype(jnp.int32).reshape(b * nj, 1, tb)
    g2 = ln_gamma.reshape(1, h)
    b2 = ln_beta.reshape(1, h)
    out = pl.pallas_call(
        _ln_body,
        grid=(b, nj),
        in_specs=[
            pl.BlockSpec((1, tb, h), lambda bi, j: (bi, j, 0)),
            pl.BlockSpec((1, 1, tb), lambda bi, j: (bi * nj + j, 0, 0)),
            pl.BlockSpec((tb, h), lambda bi, j: (j, 0)),
            pl.BlockSpec((2, h), lambda bi, j: (0, 0)),
            pl.BlockSpec((1, h), lambda bi, j: (0, 0)),
            pl.BlockSpec((1, h), lambda bi, j: (0, 0)),
        ],
        out_specs=pl.BlockSpec((1, tb, h), lambda bi, j: (bi, j, 0)),
        out_shape=jax.ShapeDtypeStruct((b, s, h), jnp.float32),
    )(tok, tid3, pos_table, type_table, g2, b2)
    return out
